# baseline (speedup 1.0000x reference)
import functools

import jax
import jax.numpy as jnp
from jax import lax
from jax.experimental import pallas as pl
from jax.experimental.pallas import tpu as pltpu
from jax.experimental.pallas import tpu_sc as plsc

N = 10000
E = 160000
F_IN = 256
H = 16
C = 16

NC = 2
NS = 16
CH = 128
CPT = 80
ICH = 16
N_CHUNKS = NS * CPT
E_PAD = N_CHUNKS * CH
N_PAD = 10112
W1_T = 32
W2_T = 32
NPT = N // NS
MCH = 125


def _prologue1_body(x_ref, w_ref, b_ref, a1_ref, ab1_ref, w2_ref, b2_ref,
                    a2_ref, ab2_ref, out_ref, par_ref):
    xn = jnp.dot(x_ref[...], w_ref[...], preferred_element_type=jnp.float32)
    xn = xn + b_ref[...]
    ar = jnp.dot(xn, a1_ref[0:H, :], preferred_element_type=jnp.float32)
    u = (ar + ab1_ref[...]) * xn
    out_ref[...] = jnp.concatenate([u, xn], axis=1)
    w2r = jnp.dot(w2_ref[...], a2_ref[0:C, :],
                  preferred_element_type=jnp.float32)
    cr = (jnp.dot(b2_ref[...], a2_ref[0:C, :],
                  preferred_element_type=jnp.float32) + ab2_ref[...])
    par_ref[...] = jnp.concatenate(
        [w2r, jnp.broadcast_to(cr, (1, 1)), jnp.zeros((7, 1), jnp.float32)],
        axis=0)


def _final_body(hc_ref, acc_ref, w2_ref, b2_ref, a2_ref, ab2_ref, out_ref):
    h = hc_ref[:, 0:H]
    cnt = hc_ref[:, H:2 * H]
    deg1 = cnt[:, 0:1] - 1.0
    S_ah = acc_ref[0:N, 0:C]
    S_h = acc_ref[0:N, C:2 * C]
    w2r = jnp.dot(w2_ref[...], a2_ref[0:C, :],
                  preferred_element_type=jnp.float32)
    cr = (jnp.dot(b2_ref[...], a2_ref[0:C, :],
                  preferred_element_type=jnp.float32) + ab2_ref[...])
    xn2 = jnp.dot(h, w2_ref[...], preferred_element_type=jnp.float32)
    xn2 = xn2 + b2_ref[...]
    ar2 = jnp.dot(xn2, a2_ref[0:C, :], preferred_element_type=jnp.float32)
    ac2 = jnp.dot(xn2, a2_ref[C:2 * C, :], preferred_element_type=jnp.float32)
    self2 = (ar2 + ac2 + ab2_ref[...]) * xn2
    S_a = jnp.dot(S_h, w2r, preferred_element_type=jnp.float32) + deg1 * cr
    Su2 = jnp.dot(S_ah, w2_ref[...],
                  preferred_element_type=jnp.float32) + S_a * b2_ref[...]
    S2p = jnp.dot(S_h, w2_ref[...],
                  preferred_element_type=jnp.float32) + deg1 * b2_ref[...]
    o = (Su2 + ac2 * S2p + self2) / cnt
    m = jnp.max(o, axis=1, keepdims=True)
    om = o - m
    lse = jnp.log(jnp.sum(jnp.exp(om), axis=1, keepdims=True))
    out_ref[...] = om - lse


def _edge_pass(tbl_s, acc, rowi, coli, s, idxr, idxc, rows0, rows1,
               sem0, sem1, deg_args=None):
    def slab(t, carry):
        base = s * CPT + t * ICH
        pltpu.sync_copy(rowi.at[pl.ds(base, ICH)], idxr)
        pltpu.sync_copy(coli.at[pl.ds(base, ICH)], idxc)
        pltpu.async_copy(tbl_s.at[idxr.at[0]], rows0, sem0)

        def body(k, carry2):
            j0 = 2 * k
            pltpu.async_copy(tbl_s.at[idxr.at[j0 + 1]], rows1, sem1)
            pltpu.make_async_copy(tbl_s.at[idxr.at[j0]], rows0, sem0).wait()
            pltpu.sync_copy(rows0, acc.at[idxc.at[j0]], add=True)
            if deg_args is not None:
                ones_v, deg_acc = deg_args
                pltpu.sync_copy(ones_v, deg_acc.at[idxc.at[j0]], add=True)

            @pl.when(j0 + 2 < ICH)
            def _():
                pltpu.async_copy(tbl_s.at[idxr.at[j0 + 2]], rows0, sem0)

            pltpu.make_async_copy(tbl_s.at[idxr.at[j0 + 1]], rows1,
                                  sem1).wait()
            pltpu.sync_copy(rows1, acc.at[idxc.at[j0 + 1]], add=True)
            if deg_args is not None:
                ones_v, deg_acc = deg_args
                pltpu.sync_copy(ones_v, deg_acc.at[idxc.at[j0 + 1]], add=True)
            return carry2

        lax.fori_loop(0, ICH // 2, body, 0)
        return carry

    lax.fori_loop(0, CPT // ICH, slab, 0)


def _make_sc_mega():
    mesh = plsc.VectorSubcoreMesh(core_axis_name="c", subcore_axis_name="s",
                                  num_cores=NC, num_subcores=NS)
    rpt = N_PAD // NS
    tpt = N // NS

    @functools.partial(
        pl.kernel,
        out_type=(
            jax.ShapeDtypeStruct((N, 2 * H), jnp.float32),
            jax.ShapeDtypeStruct((N_PAD, W2_T), jnp.float32),
        ),
        mesh=mesh,
        compiler_params=pltpu.CompilerParams(use_tc_tiling_on_sc=False,
                                             needs_layout_passes=False),
        scratch_types=[
            pltpu.VMEM((ICH, CH), jnp.int32),
            pltpu.VMEM((ICH, CH), jnp.int32),
            pltpu.VMEM((CH, W1_T), jnp.float32),
            pltpu.VMEM((CH, W1_T), jnp.float32),
            pltpu.VMEM((CH, H), jnp.float32),
            pltpu.VMEM((MCH, W1_T), jnp.float32),
            pltpu.VMEM((MCH, W1_T), jnp.float32),
            pltpu.VMEM((MCH, H), jnp.float32),
            pltpu.VMEM((MCH, W2_T), jnp.float32),
            pltpu.VMEM((MCH, 2 * H), jnp.float32),
            pltpu.VMEM((8, H), jnp.float32),
            pltpu.VMEM_SHARED((N, W1_T), jnp.float32),
            pltpu.VMEM_SHARED((N_PAD, W1_T), jnp.float32),
            pltpu.VMEM_SHARED((N_PAD, H), jnp.float32),
            pltpu.VMEM_SHARED((N, W2_T), jnp.float32),
            pltpu.VMEM_SHARED((N_PAD, W2_T), jnp.float32),
            pltpu.SemaphoreType.DMA,
            pltpu.SemaphoreType.DMA,
        ],
    )
    def sc_mega(table1, rowi, coli, zeros1, zeros2, zeros16, onesh, params,
                hc_out, acc2_out,
                idxr, idxc, rows0, rows1, ones_v,
                macc, mtbl, mdeg, mt2, mhc, par,
                tbl1_s, acc1, deg_s, tbl2_s, acc2, sem0, sem1):
        c = lax.axis_index("c")
        s = lax.axis_index("s")

        @pl.when(c == 0)
        def _work():
            pltpu.sync_copy(zeros1.at[pl.ds(s * rpt, rpt)],
                            acc1.at[pl.ds(s * rpt, rpt)])
            pltpu.sync_copy(zeros2.at[pl.ds(s * rpt, rpt)],
                            acc2.at[pl.ds(s * rpt, rpt)])
            pltpu.sync_copy(zeros16.at[pl.ds(s * rpt, rpt)],
                            deg_s.at[pl.ds(s * rpt, rpt)])
            pltpu.sync_copy(table1.at[pl.ds(s * tpt, tpt)],
                            tbl1_s.at[pl.ds(s * tpt, tpt)])
            pltpu.sync_copy(onesh, ones_v)
            pltpu.sync_copy(params, par)
            plsc.subcore_barrier()

            _edge_pass(tbl1_s, acc1, rowi, coli, s, idxr, idxc, rows0, rows1,
                       sem0, sem1, deg_args=(ones_v, deg_s))
            plsc.subcore_barrier()

            a1r = par[0, :]
            a1c = par[1, :]
            w2r = par[2, :]
            ab1 = par[3, :]
            crv = par[4, :]

            def slab(t, carry):
                base = s * tpt + t * MCH
                pltpu.sync_copy(acc1.at[pl.ds(base, MCH)], macc)
                pltpu.sync_copy(tbl1_s.at[pl.ds(base, MCH)], mtbl)
                pltpu.sync_copy(deg_s.at[pl.ds(base, MCH)], mdeg)

                def node(j, carry2):
                    S1 = macc[j, 0:H]
                    S2 = macc[j, H:2 * H]
                    deg = mdeg[j, 0:H]
                    xn1 = mtbl[j, H:2 * H]
                    ar1 = jnp.sum(xn1 * a1r)
                    ac1 = jnp.sum(xn1 * a1c)
                    cnt = deg + 1.0
                    self1 = (ar1 + ac1) * xn1 + ab1 * xn1
                    h = jnp.maximum((S1 + ac1 * S2 + self1) / cnt, 0.0)
                    alpha = jnp.sum(h * w2r) + crv
                    mt2[j, 0:H] = alpha * h
                    mt2[j, H:2 * H] = h
                    mhc[j, 0:H] = h
                    mhc[j, H:2 * H] = cnt
                    return carry2

                lax.fori_loop(0, MCH, node, 0)
                pltpu.sync_copy(mt2, tbl2_s.at[pl.ds(base, MCH)])
                pltpu.sync_copy(mhc, hc_out.at[pl.ds(base, MCH)])
                return carry

            lax.fori_loop(0, NPT // MCH, slab, 0)
            plsc.subcore_barrier()

            _edge_pass(tbl2_s, acc2, rowi, coli, s, idxr, idxc,
                       rows0, rows1, sem0, sem1)
            plsc.subcore_barrier()
            pltpu.sync_copy(acc2.at[pl.ds(s * rpt, rpt)],
                            acc2_out.at[pl.ds(s * rpt, rpt)])

    return sc_mega


_make_sc_mega = functools.lru_cache(maxsize=None)(_make_sc_mega)


@jax.jit
def kernel(x, edge_index, W1, b1, A1, Ab1, W2, b2, A2, Ab2):
    f32 = jnp.float32
    rowi = jnp.concatenate(
        [edge_index[0], jnp.zeros((E_PAD - E,), jnp.int32)]
    ).reshape(N_CHUNKS, CH)
    coli = jnp.concatenate(
        [edge_index[1], jnp.full((E_PAD - E,), N, jnp.int32)]
    ).reshape(N_CHUNKS, CH)

    table1, w2rcr = pl.pallas_call(
        _prologue1_body,
        out_shape=(
            jax.ShapeDtypeStruct((N, W1_T), f32),
            jax.ShapeDtypeStruct((24, 1), f32),
        ),
    )(x, W1, b1.reshape(1, H), A1, Ab1.reshape(1, 1), W2, b2.reshape(1, C),
      A2, Ab2.reshape(1, 1))
    params = jnp.stack([
        A1[0:H, 0], A1[H:2 * H, 0], w2rcr[0:C, 0],
        jnp.broadcast_to(Ab1, (H,)), jnp.broadcast_to(w2rcr[C, 0], (H,)),
        jnp.zeros((H,), f32), jnp.zeros((H,), f32), jnp.zeros((H,), f32),
    ], axis=0)

    zeros1 = jnp.zeros((N_PAD, W1_T), f32)
    zeros2 = jnp.zeros((N_PAD, W2_T), f32)
    zeros16 = jnp.zeros((N_PAD, H), f32)
    onesh = jnp.ones((CH, H), f32)
    hc, acc2 = _make_sc_mega()(table1, rowi, coli, zeros1, zeros2, zeros16,
                               onesh, params)

    out = pl.pallas_call(
        _final_body,
        out_shape=jax.ShapeDtypeStruct((N, C), f32),
    )(hc, acc2, W2, b2.reshape(1, C), A2, Ab2.reshape(1, 1))
    return out


# speedup vs baseline: 1.0995x; 1.0995x over previous
import functools

import jax
import jax.numpy as jnp
from jax import lax
from jax.experimental import pallas as pl
from jax.experimental.pallas import tpu as pltpu
from jax.experimental.pallas import tpu_sc as plsc

N = 10000
E = 160000
F_IN = 256
H = 16
C = 16

NC = 2
NS = 16
NW = NC * NS
CH = 128
CPW0 = 24
CPW1 = 56
N_CHUNKS = NS * (CPW0 + CPW1)
E_PAD = N_CHUNKS * CH
N_PAD = 10112
W1_T = 48
W2_T = 32


def _prologue1_body(x_ref, w_ref, b_ref, a_ref, ab_ref, out_ref):
    xn = jnp.dot(x_ref[...], w_ref[...], preferred_element_type=jnp.float32)
    xn = xn + b_ref[...]
    ar = jnp.dot(xn, a_ref[0:H, :], preferred_element_type=jnp.float32)
    u = (ar + ab_ref[...]) * xn
    out_ref[...] = jnp.concatenate([u, xn, jnp.ones_like(xn)], axis=1)


def _mid_body(acc_ref, t1_ref, a1_ref, ab1_ref, w2_ref, b2_ref, a2_ref,
              ab2_ref, t2_ref, cnt_ref):
    S = (acc_ref[0] + acc_ref[1])[0:N, :]
    S1 = S[:, 0:H]
    S2 = S[:, H:2 * H]
    deg = S[:, 2 * H:3 * H]
    xn1 = t1_ref[:, H:2 * H]
    ar1 = jnp.dot(xn1, a1_ref[0:H, :], preferred_element_type=jnp.float32)
    ac1 = jnp.dot(xn1, a1_ref[H:2 * H, :], preferred_element_type=jnp.float32)
    self1 = (ar1 + ac1 + ab1_ref[...]) * xn1
    cnt = deg + 1.0
    h = jnp.maximum((S1 + ac1 * S2 + self1) / cnt, 0.0)
    xn2 = jnp.dot(h, w2_ref[...], preferred_element_type=jnp.float32)
    xn2 = xn2 + b2_ref[...]
    ar2 = jnp.dot(xn2, a2_ref[0:C, :], preferred_element_type=jnp.float32)
    u2 = (ar2 + ab2_ref[...]) * xn2
    t2_ref[...] = jnp.concatenate([u2, xn2], axis=1)
    cnt_ref[...] = cnt


def _final_body(acc_ref, t2_ref, cnt_ref, a2_ref, ab2_ref, out_ref):
    S = (acc_ref[0] + acc_ref[1])[0:N, :]
    S1 = S[:, 0:C]
    S2 = S[:, C:2 * C]
    xn2 = t2_ref[:, C:2 * C]
    ar2 = jnp.dot(xn2, a2_ref[0:C, :], preferred_element_type=jnp.float32)
    ac2 = jnp.dot(xn2, a2_ref[C:2 * C, :], preferred_element_type=jnp.float32)
    self2 = (ar2 + ac2 + ab2_ref[...]) * xn2
    o = (S1 + ac2 * S2 + self2) / cnt_ref[...]
    m = jnp.max(o, axis=1, keepdims=True)
    om = o - m
    lse = jnp.log(jnp.sum(jnp.exp(om), axis=1, keepdims=True))
    out_ref[...] = om - lse


def _make_sc_pass(width):
    mesh = plsc.VectorSubcoreMesh(core_axis_name="c", subcore_axis_name="s",
                                  num_cores=NC, num_subcores=NS)
    rpt = N_PAD // NS

    @functools.partial(
        pl.kernel,
        out_type=jax.ShapeDtypeStruct((NC, N_PAD, width), jnp.float32),
        mesh=mesh,
        compiler_params=pltpu.CompilerParams(use_tc_tiling_on_sc=False),
        scratch_types=[
            pltpu.VMEM((CPW1, CH), jnp.int32),
            pltpu.VMEM((CPW1, CH), jnp.int32),
            pltpu.VMEM((CH, width), jnp.float32),
            pltpu.VMEM((CH, width), jnp.float32),
            pltpu.VMEM_SHARED((N_PAD, width), jnp.float32),
            pltpu.VMEM_SHARED((N, width), jnp.float32),
            pltpu.SemaphoreType.DMA,
            pltpu.SemaphoreType.DMA,
        ],
    )
    def sc_pass(table, rowi, coli, zeros, out, idxr, idxc, rows0, rows1,
                acc, tbl_s, sem0, sem1):
        c = lax.axis_index("c")
        s = lax.axis_index("s")
        nch = jnp.where(c == 0, CPW0, CPW1)
        base = jnp.where(c == 0, s * CPW0, NS * CPW0 + s * CPW1)
        pltpu.sync_copy(zeros.at[pl.ds(s * rpt, rpt)],
                        acc.at[pl.ds(s * rpt, rpt)])
        tpt = N // NS
        pltpu.sync_copy(table.at[pl.ds(s * tpt, tpt)],
                        tbl_s.at[pl.ds(s * tpt, tpt)])
        pltpu.sync_copy(rowi.at[pl.ds(base, CPW0)], idxr.at[pl.ds(0, CPW0)])

        @pl.when(c == 1)
        def _():
            pltpu.sync_copy(rowi.at[pl.ds(base + CPW0, CPW1 - CPW0)],
                            idxr.at[pl.ds(CPW0, CPW1 - CPW0)])
        pltpu.sync_copy(coli.at[pl.ds(base, CPW0)], idxc.at[pl.ds(0, CPW0)])

        @pl.when(c == 1)
        def _():
            pltpu.sync_copy(coli.at[pl.ds(base + CPW0, CPW1 - CPW0)],
                            idxc.at[pl.ds(CPW0, CPW1 - CPW0)])
        plsc.subcore_barrier()

        pltpu.async_copy(tbl_s.at[idxr.at[0]], rows0, sem0)

        def body(k, carry):
            j0 = 2 * k
            pltpu.async_copy(tbl_s.at[idxr.at[j0 + 1]], rows1, sem1)
            pltpu.make_async_copy(tbl_s.at[idxr.at[j0]], rows0, sem0).wait()
            pltpu.sync_copy(rows0, acc.at[idxc.at[j0]], add=True)

            @pl.when(j0 + 2 < nch)
            def _():
                pltpu.async_copy(tbl_s.at[idxr.at[j0 + 2]], rows0, sem0)

            pltpu.make_async_copy(
                tbl_s.at[idxr.at[j0 + 1]], rows1, sem1).wait()
            pltpu.sync_copy(rows1, acc.at[idxc.at[j0 + 1]], add=True)
            return carry

        lax.fori_loop(0, nch // 2, body, 0)
        plsc.subcore_barrier()
        pltpu.sync_copy(acc.at[pl.ds(s * rpt, rpt)],
                        out.at[c, pl.ds(s * rpt, rpt)])

    return sc_pass


_make_sc_pass = functools.lru_cache(maxsize=None)(_make_sc_pass)


@jax.jit
def kernel(x, edge_index, W1, b1, A1, Ab1, W2, b2, A2, Ab2):
    f32 = jnp.float32
    rowi = jnp.concatenate(
        [edge_index[0], jnp.zeros((E_PAD - E,), jnp.int32)]
    ).reshape(N_CHUNKS, CH)
    coli = jnp.concatenate(
        [edge_index[1], jnp.full((E_PAD - E,), N, jnp.int32)]
    ).reshape(N_CHUNKS, CH)

    table1 = pl.pallas_call(
        _prologue1_body,
        out_shape=jax.ShapeDtypeStruct((N, W1_T), f32),
    )(x, W1, b1.reshape(1, H), A1, Ab1.reshape(1, 1))

    zeros1 = jnp.zeros((N_PAD, W1_T), f32)
    acc1 = _make_sc_pass(W1_T)(table1, rowi, coli, zeros1)

    table2, cnt = pl.pallas_call(
        _mid_body,
        out_shape=(
            jax.ShapeDtypeStruct((N, W2_T), f32),
            jax.ShapeDtypeStruct((N, C), f32),
        ),
    )(acc1, table1, A1, Ab1.reshape(1, 1), W2, b2.reshape(1, C), A2,
      Ab2.reshape(1, 1))

    zeros2 = jnp.zeros((N_PAD, W2_T), f32)
    acc2 = _make_sc_pass(W2_T)(table2, rowi, coli, zeros2)

    out = pl.pallas_call(
        _final_body,
        out_shape=jax.ShapeDtypeStruct((N, C), f32),
    )(acc2, table2, cnt, A2, Ab2.reshape(1, 1))
    return out
